# baseline (speedup 1.0000x reference)
import functools

import numpy as np
import jax
import jax.numpy as jnp
from jax import lax
from jax.experimental import pallas as pl
from jax.experimental.pallas import tpu as pltpu
from jax.experimental.pallas import tpu_sc as plsc

VOCAB = 100000
D_MODEL = 1024
MAX_LEN = 2048
BATCH = 4
SEQ = 2048

NC, NS = 2, 16
NW = NC * NS
LANES = 16
TOTAL = BATCH * SEQ
PER_W = TOTAL // NW
CHUNK = 16
N_CHUNKS = PER_W // CHUNK
NBUF = 4
NOBUF = 2
SCALE = float(D_MODEL) ** 0.5
PE_WORDS = D_MODEL // 2


def _make_pe(max_len, d_model):
    pe = np.zeros((max_len, d_model), dtype=np.float32)
    position = np.arange(0, max_len, dtype=np.float32)[:, None]
    div_term = np.exp(
        np.arange(0, d_model, 2, dtype=np.float32) * -(np.log(10000.0) / d_model))
    pe[:, 0::2] = np.sin(position * div_term)
    pe[:, 1::2] = np.cos(position * div_term)
    return pe


def _pack_pe(pe):
    bits = pe.view(np.uint32)
    bf = ((bits + 0x7FFF + ((bits >> 16) & 1)) >> 16).astype(np.uint32)
    g = bf.reshape(pe.shape[0], D_MODEL // 32, 2, 16)
    words = g[:, :, 0, :] | (g[:, :, 1, :] << 16)
    return words.reshape(pe.shape[0], PE_WORDS).view(np.int32)


_PE_PACKED = _pack_pe(_make_pe(MAX_LEN, D_MODEL))


def _sc_embed(x_flat, table, pe_pk):
    mesh = plsc.VectorSubcoreMesh(core_axis_name="c", subcore_axis_name="s")

    @functools.partial(
        pl.kernel,
        out_type=jax.ShapeDtypeStruct((TOTAL, D_MODEL), jnp.float32),
        mesh=mesh,
        scratch_types=[
            pltpu.VMEM((PER_W,), jnp.int32),
            pltpu.VMEM((NBUF, CHUNK, D_MODEL), jnp.float32),
            pltpu.VMEM((NOBUF, CHUNK, D_MODEL), jnp.float32),
            pltpu.VMEM((NOBUF, CHUNK, PE_WORDS), jnp.int32),
            pltpu.SemaphoreType.DMA((NBUF,)),
            pltpu.SemaphoreType.DMA((NOBUF,)),
            pltpu.SemaphoreType.DMA((NOBUF,)),
        ],
    )
    def k(x_hbm, table_hbm, pe_hbm, out_hbm,
          idx_v, rows_v, obuf_v, peb_v, gsem, psem, osem):
        wid = lax.axis_index("s") * NC + lax.axis_index("c")
        base = wid * PER_W
        s0 = base % SEQ

        pltpu.sync_copy(x_hbm.at[pl.ds(base, PER_W)], idx_v)

        def fire_gather(c, b):
            pltpu.async_copy(
                table_hbm.at[idx_v.at[pl.ds(c * CHUNK, CHUNK)]],
                rows_v.at[b], gsem.at[b])

        def fire_pe(c, b):
            pltpu.async_copy(
                pe_hbm.at[pl.ds(s0 + c * CHUNK, CHUNK)],
                peb_v.at[b], psem.at[b])

        for b in range(NBUF):
            fire_gather(b, b)
        for b in range(NOBUF):
            fire_pe(b, b)

        def chunk_body(c, _):
            b4 = lax.rem(c, NBUF)
            b2 = lax.rem(c, NOBUF)
            pltpu.make_async_copy(
                table_hbm.at[idx_v.at[pl.ds(c * CHUNK, CHUNK)]],
                rows_v.at[b4], gsem.at[b4]).wait()
            pltpu.make_async_copy(
                pe_hbm.at[pl.ds(s0 + c * CHUNK, CHUNK)],
                peb_v.at[b2], psem.at[b2]).wait()

            @pl.when(c >= NOBUF)
            def _():
                pltpu.make_async_copy(
                    obuf_v.at[b2],
                    out_hbm.at[pl.ds(base + (c - NOBUF) * CHUNK, CHUNK)],
                    osem.at[b2]).wait()

            def row_body(r, _):
                for g in range(D_MODEL // 32):
                    w = peb_v[b2, r, pl.ds(g * 16, 16)]
                    lo = lax.bitcast_convert_type(w << 16, jnp.float32)
                    hi = lax.bitcast_convert_type(w & jnp.int32(-65536), jnp.float32)
                    sl0 = pl.ds(g * 32, 16)
                    sl1 = pl.ds(g * 32 + 16, 16)
                    obuf_v[b2, r, sl0] = rows_v[b4, r, sl0] * SCALE + lo
                    obuf_v[b2, r, sl1] = rows_v[b4, r, sl1] * SCALE + hi
                return 0

            lax.fori_loop(0, CHUNK, row_body, 0)

            pltpu.async_copy(
                obuf_v.at[b2],
                out_hbm.at[pl.ds(base + c * CHUNK, CHUNK)],
                osem.at[b2])

            @pl.when(c < N_CHUNKS - NBUF)
            def _():
                fire_gather(c + NBUF, b4)

            @pl.when(c < N_CHUNKS - NOBUF)
            def _():
                fire_pe(c + NOBUF, b2)

            return 0

        lax.fori_loop(0, N_CHUNKS, chunk_body, 0)

        for i in range(NOBUF):
            c = N_CHUNKS - NOBUF + i
            pltpu.make_async_copy(
                obuf_v.at[c % NOBUF],
                out_hbm.at[pl.ds(base + c * CHUNK, CHUNK)],
                osem.at[c % NOBUF]).wait()

    return k(x_flat, table, pe_pk)


def kernel(x, table):
    x_flat = jnp.reshape(x, (TOTAL,)).astype(jnp.int32)
    out = _sc_embed(x_flat, table, _PE_PACKED)
    return jnp.reshape(out, (BATCH, SEQ, D_MODEL))


# speedup vs baseline: 1.5862x; 1.5862x over previous
import functools

import numpy as np
import jax
import jax.numpy as jnp
from jax import lax
from jax.experimental import pallas as pl
from jax.experimental.pallas import tpu as pltpu
from jax.experimental.pallas import tpu_sc as plsc

VOCAB = 100000
D_MODEL = 1024
MAX_LEN = 2048
BATCH = 4
SEQ = 2048

NC, NS = 2, 16
NW = NC * NS
TOTAL = BATCH * SEQ
PER_W = TOTAL // NW
CHUNK = 16
N_CHUNKS = PER_W // CHUNK
NBUF = 4
GLEAD = 2
SCALE = float(D_MODEL) ** 0.5
PE_WORDS = D_MODEL // 2


def _make_pe(max_len, d_model):
    pe = np.zeros((max_len, d_model), dtype=np.float32)
    position = np.arange(0, max_len, dtype=np.float32)[:, None]
    div_term = np.exp(
        np.arange(0, d_model, 2, dtype=np.float32) * -(np.log(10000.0) / d_model))
    pe[:, 0::2] = np.sin(position * div_term)
    pe[:, 1::2] = np.cos(position * div_term)
    return pe


def _pack_pe(pe):
    bits = pe.view(np.uint32)
    bf = ((bits + 0x7FFF + ((bits >> 16) & 1)) >> 16).astype(np.uint32)
    g = bf.reshape(pe.shape[0], D_MODEL // 32, 2, 16)
    words = g[:, :, 0, :] | (g[:, :, 1, :] << 16)
    return words.reshape(pe.shape[0], PE_WORDS).view(np.int32)


_PE_PACKED = _pack_pe(_make_pe(MAX_LEN, D_MODEL))


def _sc_embed(x_flat, table, pe_pk):
    mesh = plsc.VectorSubcoreMesh(core_axis_name="c", subcore_axis_name="s")

    @functools.partial(
        pl.kernel,
        out_type=jax.ShapeDtypeStruct((TOTAL, D_MODEL), jnp.float32),
        mesh=mesh,
        scratch_types=[
            pltpu.VMEM((PER_W,), jnp.int32),
            [pltpu.VMEM((CHUNK, D_MODEL), jnp.float32) for _ in range(NBUF)],
            [pltpu.VMEM((CHUNK, PE_WORDS), jnp.int32) for _ in range(2)],
            [pltpu.SemaphoreType.DMA for _ in range(NBUF)],
            [pltpu.SemaphoreType.DMA for _ in range(2)],
            [pltpu.SemaphoreType.DMA for _ in range(NBUF)],
        ],
    )
    def k(x_hbm, table_hbm, pe_hbm, out_hbm,
          idx_v, rows, pebs, gsems, psems, osems):
        wid = lax.axis_index("s") * NC + lax.axis_index("c")
        base = wid * PER_W
        s0 = base % SEQ

        pltpu.sync_copy(x_hbm.at[pl.ds(base, PER_W)], idx_v)

        def fire_gather(c, b):
            pltpu.async_copy(
                table_hbm.at[idx_v.at[pl.ds(c * CHUNK, CHUNK)]],
                rows[b], gsems[b])

        def wait_gather(c, b):
            pltpu.make_async_copy(
                table_hbm.at[idx_v.at[pl.ds(c * CHUNK, CHUNK)]],
                rows[b], gsems[b]).wait()

        def fire_pe(c, b):
            pltpu.async_copy(
                pe_hbm.at[pl.ds(s0 + c * CHUNK, CHUNK)], pebs[b], psems[b])

        def wait_pe(c, b):
            pltpu.make_async_copy(
                pe_hbm.at[pl.ds(s0 + c * CHUNK, CHUNK)], pebs[b],
                psems[b]).wait()

        def fire_out(c, b):
            pltpu.async_copy(
                rows[b], out_hbm.at[pl.ds(base + c * CHUNK, CHUNK)], osems[b])

        def wait_out(c, b):
            pltpu.make_async_copy(
                rows[b], out_hbm.at[pl.ds(base + c * CHUNK, CHUNK)],
                osems[b]).wait()

        for c in range(GLEAD):
            fire_gather(c, c)
            fire_pe(c, c)

        def super_body(g, _):
            c0 = g * NBUF
            for j in range(NBUF):
                c = c0 + j
                bp = j % 2
                wait_gather(c, j)
                wait_pe(c, bp)

                def row_body(r, _):
                    for q in range(D_MODEL // 32):
                        w = pebs[bp][r, pl.ds(q * 16, 16)]
                        lo = lax.bitcast_convert_type(w << 16, jnp.float32)
                        hi = lax.bitcast_convert_type(
                            w & jnp.int32(-65536), jnp.float32)
                        sl0 = pl.ds(q * 32, 16)
                        sl1 = pl.ds(q * 32 + 16, 16)
                        rows[j][r, sl0] = rows[j][r, sl0] * SCALE + lo
                        rows[j][r, sl1] = rows[j][r, sl1] * SCALE + hi
                    return 0

                lax.fori_loop(0, CHUNK, row_body, 0)
                fire_out(c, j)

                @pl.when(c + GLEAD < N_CHUNKS)
                def _():
                    nb = (j + GLEAD) % NBUF

                    @pl.when(c >= GLEAD)
                    def _():
                        wait_out(c - GLEAD, nb)

                    fire_gather(c + GLEAD, nb)
                    fire_pe(c + GLEAD, bp)
            return 0

        lax.fori_loop(0, N_CHUNKS // NBUF, super_body, 0)

        for c in range(N_CHUNKS - GLEAD, N_CHUNKS):
            wait_out(c, c % NBUF)

    return k(x_flat, table, pe_pk)


def kernel(x, table):
    x_flat = jnp.reshape(x, (TOTAL,)).astype(jnp.int32)
    out = _sc_embed(x_flat, table, _PE_PACKED)
    return jnp.reshape(out, (BATCH, SEQ, D_MODEL))
